# baseline (speedup 1.0000x reference)
import jax
import jax.numpy as jnp
from jax.experimental import pallas as pl
from jax.experimental.pallas import tpu as pltpu


def _sigmoid_body(logits_ref, prob_ref):
    prob_ref[...] = jax.nn.sigmoid(logits_ref[...])


def _pallas_sigmoid(logits):
    bs = logits.shape[0]
    return pl.pallas_call(
        _sigmoid_body,
        out_shape=jax.ShapeDtypeStruct(logits.shape, jnp.float32),
        grid=(bs,),
        in_specs=[pl.BlockSpec((1,) + logits.shape[1:], lambda b: (b, 0, 0))],
        out_specs=pl.BlockSpec((1,) + logits.shape[1:], lambda b: (b, 0, 0)),
    )(logits)


def kernel(pred_logits, pred_boxes, target_sizes):
    bs = pred_logits.shape[0]
    num_classes = pred_logits.shape[-1]
    out_logits = pred_logits.reshape(bs, -1, num_classes)
    out_boxes = pred_boxes.reshape(bs, -1, 4)
    num_queries = out_logits.shape[1]
    prob = _pallas_sigmoid(out_logits)
    topk_values, topk_indexes = jax.lax.top_k(prob.reshape(bs, -1), num_queries)
    scores = topk_values
    topk_boxes = topk_indexes // num_classes
    labels = jnp.ones_like(topk_indexes)
    cx, cy, w, h = (out_boxes[..., 0], out_boxes[..., 1],
                    out_boxes[..., 2], out_boxes[..., 3])
    boxes = jnp.stack([cx - 0.5 * w, cy - 0.5 * h,
                       cx + 0.5 * w, cy + 0.5 * h], axis=-1)
    gather_idx = jnp.repeat(topk_boxes[:, :, None], 4, axis=-1)
    boxes = jnp.take_along_axis(boxes, gather_idx, axis=1)
    img_h = target_sizes[:, 0]
    img_w = target_sizes[:, 1]
    scale_fct = jnp.stack([img_w, img_h, img_w, img_h], axis=1)
    boxes = boxes * scale_fct[:, None, :]
    return scores, labels, boxes


# speedup vs baseline: 13.9279x; 13.9279x over previous
import functools

import jax
import jax.numpy as jnp
from jax import lax
from jax.experimental import pallas as pl
from jax.experimental.pallas import tpu as pltpu
from jax.experimental.pallas import tpu_sc as plsc

BS = 16
NQ = 20000
NC = 91
NQP = 20480
NF = NQP * NC
NH = NF // 2
K = NQ
HIST = 16384
CAP_HALF = 14336
CAPC = 2 * CAP_HALF
CH = 29120
NCHUNK = NH // CH
LANES = 16

_mesh = plsc.VectorSubcoreMesh(core_axis_name="c", subcore_axis_name="s")
_sc_params = pltpu.CompilerParams(needs_layout_passes=False)


def _sigmoid_pad_body(logits_ref, prob_ref):
    prob_ref[0, :NQ, :] = jax.nn.sigmoid(logits_ref[0, :, :])
    prob_ref[0, NQ:, :] = jnp.zeros((NQP - NQ, NC), jnp.float32)


def _sigmoid_pad(logits3):
    return pl.pallas_call(
        _sigmoid_pad_body,
        out_shape=jax.ShapeDtypeStruct((BS, NQP, NC), jnp.float32),
        grid=(BS,),
        in_specs=[pl.BlockSpec((1, NQ, NC), lambda b: (b, 0, 0))],
        out_specs=pl.BlockSpec((1, NQP, NC), lambda b: (b, 0, 0)),
    )(logits3)


@functools.partial(
    pl.kernel,
    out_type=(
        jax.ShapeDtypeStruct((BS * CAPC,), jnp.int32),
        jax.ShapeDtypeStruct((BS * CAPC,), jnp.int32),
        jax.ShapeDtypeStruct((BS * 16,), jnp.int32),
    ),
    mesh=_mesh,
    compiler_params=_sc_params,
    scratch_types=[
        pltpu.VMEM((CH,), jnp.float32),
        pltpu.VMEM((HIST,), jnp.int32),
        pltpu.VMEM((HIST,), jnp.int32),
        pltpu.VMEM((CAP_HALF + 16,), jnp.int32),
        pltpu.VMEM((CAP_HALF + 16,), jnp.int32),
        pltpu.VMEM((16,), jnp.int32),
        pltpu.VMEM_SHARED((16, HIST), jnp.int32),
    ],
)
def _select_kernel(prob_hbm, ckey_hbm, cidx_hbm, cnt_hbm,
                   chunk, hist, phist, skey, sidx, crow, shist):
    c = lax.axis_index("c")
    s = lax.axis_index("s")
    b = c * 8 + s // 2
    h = s % 2
    base = b * NF + h * NH
    ones = jnp.full((LANES,), 1, jnp.int32)
    iota = lax.iota(jnp.int32, LANES)

    def _zero_hist(i, _):
        hist[pl.ds(i * LANES, LANES)] = jnp.zeros((LANES,), jnp.int32)
        return 0
    lax.fori_loop(0, HIST // LANES, _zero_hist, 0)

    def _hist_chunk(g, _):
        pltpu.sync_copy(prob_hbm.at[pl.ds(base + g * CH, CH)], chunk)

        def _hist_vreg(j, _):
            v = chunk[pl.ds(j * LANES, LANES)]
            key = plsc.bitcast(v, jnp.int32)
            d = key >> 16
            plsc.addupdate_scatter(hist, [d], ones)
            return 0
        lax.fori_loop(0, CH // LANES, _hist_vreg, 0)
        return 0
    lax.fori_loop(0, NCHUNK, _hist_chunk, 0)

    pltpu.sync_copy(hist, shist.at[s])
    plsc.subcore_barrier()
    pltpu.sync_copy(shist.at[s + 1 - 2 * h], phist)

    zero_v = jnp.zeros((LANES,), jnp.int32)
    i15 = jnp.full((LANES,), 15, jnp.int32)

    def _thresh(i, carry):
        tot, tot_own, bstar, ctot, cown, found = carry
        jj = HIST // LANES - 1 - i
        vo = hist[pl.ds(jj * LANES, LANES)]
        vp = phist[pl.ds(jj * LANES, LANES)]
        ro = lax.rev(vo, (0,))
        rt = lax.rev(vo + vp, (0,))
        cso = plsc.cumsum(ro)
        cst = plsc.cumsum(rt)
        t = cst + tot
        m = t >= K
        npop = plsc.all_reduce_population_count(m)
        ffs = plsc.all_reduce_ffs(m)
        upd = (npop > 0) & jnp.logical_not(found)
        ffs_c = jnp.where(npop > 0, ffs, zero_v)
        sel_b = jj * LANES + 15 - ffs_c
        sel_ctot = jnp.take(t, ffs_c)
        sel_cown = jnp.take(cso, ffs_c) + tot_own
        bstar = jnp.where(upd, sel_b, bstar)
        ctot = jnp.where(upd, sel_ctot, ctot)
        cown = jnp.where(upd, sel_cown, cown)
        found = found | (npop > 0)
        tot = tot + jnp.take(cst, i15)
        tot_own = tot_own + jnp.take(cso, i15)
        return tot, tot_own, bstar, ctot, cown, found

    init = (zero_v, zero_v, zero_v, zero_v, zero_v,
            jnp.zeros((LANES,), jnp.bool_))
    _, _, bstar, ctot, cown, _ = lax.fori_loop(0, HIST // LANES, _thresh, init)
    tkey = bstar << 16

    c0v = jnp.where(h == 0, cown, ctot - cown)
    c1v = ctot - c0v

    @pl.when(h == 0)
    def _():
        crow[...] = jnp.where(iota == 0, c0v,
                              jnp.where(iota == 1, c1v, zero_v))
        pltpu.sync_copy(crow, cnt_hbm.at[pl.ds(16 * b, 16)])

    def _compact_chunk(g, ptr):
        pltpu.sync_copy(prob_hbm.at[pl.ds(base + g * CH, CH)], chunk)

        def _compact_vreg(j, ptr):
            v = chunk[pl.ds(j * LANES, LANES)]
            key = plsc.bitcast(v, jnp.int32)
            m = (key >= tkey) & (ptr < CAP_HALF)
            lvec = iota + (h * NH + g * CH + j * LANES)
            plsc.store_compressed(skey.at[pl.ds(ptr, LANES)], key, mask=m)
            plsc.store_compressed(sidx.at[pl.ds(ptr, LANES)], lvec, mask=m)
            return ptr + jnp.sum(m.astype(jnp.int32))
        return lax.fori_loop(0, CH // LANES, _compact_vreg, ptr)

    lax.fori_loop(0, NCHUNK, _compact_chunk, jnp.int32(0))

    pltpu.sync_copy(skey.at[pl.ds(0, CAP_HALF)],
                    ckey_hbm.at[pl.ds(b * CAPC + h * CAP_HALF, CAP_HALF)])
    pltpu.sync_copy(sidx.at[pl.ds(0, CAP_HALF)],
                    cidx_hbm.at[pl.ds(b * CAPC + h * CAP_HALF, CAP_HALF)])


RADIX = 1024


@functools.partial(
    pl.kernel,
    out_type=(
        jax.ShapeDtypeStruct((BS * NQ,), jnp.int32),
        jax.ShapeDtypeStruct((BS * NQ,), jnp.int32),
    ),
    mesh=_mesh,
    compiler_params=_sc_params,
    scratch_types=[
        pltpu.VMEM((CAPC,), jnp.int32),
        pltpu.VMEM((CAPC,), jnp.int32),
        pltpu.VMEM((CAPC,), jnp.int32),
        pltpu.VMEM((CAPC,), jnp.int32),
        pltpu.VMEM((RADIX,), jnp.int32),
        pltpu.VMEM((16,), jnp.int32),
    ],
)
def _sort_kernel(ckey_hbm, cidx_hbm, cnt_hbm, score_hbm, qidx_hbm,
                 ka, pa, kb, pb, offs, crow):
    c = lax.axis_index("c")
    s = lax.axis_index("s")
    active = s < 8
    b = c * 8 + jnp.where(active, s, 0)
    ones = jnp.full((LANES,), 1, jnp.int32)
    iota = lax.iota(jnp.int32, LANES)
    nv = CAPC // LANES

    @pl.when(active)
    def _():
        pltpu.sync_copy(ckey_hbm.at[pl.ds(b * CAPC, CAPC)], ka)
        pltpu.sync_copy(cidx_hbm.at[pl.ds(b * CAPC, CAPC)], pa)
        pltpu.sync_copy(cnt_hbm.at[pl.ds(16 * b, 16)], crow)
        cv = crow[...]
        c0 = jnp.take(cv, jnp.zeros((LANES,), jnp.int32))
        c1 = jnp.take(cv, jnp.full((LANES,), 1, jnp.int32))

        for p in range(3):
            src_k, src_p = (ka, pa) if p % 2 == 0 else (kb, pb)
            dst_k, dst_p = (kb, pb) if p % 2 == 0 else (ka, pa)
            shift = 10 * p

            def _zero(i, _):
                offs[pl.ds(i * LANES, LANES)] = jnp.zeros((LANES,), jnp.int32)
                return 0
            lax.fori_loop(0, RADIX // LANES, _zero, 0)

            def _load_key(j):
                kv = src_k[pl.ds(j * LANES, LANES)]
                if p == 0:
                    pos = iota + j * LANES
                    valid = (pos < c0) | ((pos >= CAP_HALF)
                                          & (pos < CAP_HALF + c1))
                    kv = jnp.where(valid, kv, 0)
                return kv

            def _hist(j, _):
                kv = _load_key(j)
                dd = (jnp.bitwise_not(kv) >> shift) & (RADIX - 1)
                plsc.addupdate_scatter(offs, [dd], ones)
                return 0
            lax.fori_loop(0, nv, _hist, 0)

            def _scan(i, carry):
                v = offs[pl.ds(i * LANES, LANES)]
                cs = plsc.cumsum(v)
                offs[pl.ds(i * LANES, LANES)] = cs - v + carry
                return carry + jnp.take(cs, jnp.full((LANES,), 15, jnp.int32))
            lax.fori_loop(0, RADIX // LANES, _scan,
                          jnp.zeros((LANES,), jnp.int32))

            def _permute(j, _):
                kv = _load_key(j)
                pv = src_p[pl.ds(j * LANES, LANES)]
                dd = (jnp.bitwise_not(kv) >> shift) & (RADIX - 1)
                cnt, _last = plsc.scan_count(dd)
                basev = plsc.load_gather(offs, [dd])
                pos = basev + cnt - 1
                plsc.store_scatter(dst_k, [pos], kv)
                plsc.store_scatter(dst_p, [pos], pv)
                plsc.addupdate_scatter(offs, [dd], ones)
                return 0
            lax.fori_loop(0, nv, _permute, 0)

        inv91 = jnp.float32(1.0 / 91.0)

        def _qidx(j, _):
            pv = pb[pl.ds(j * LANES, LANES)]
            q = ((pv.astype(jnp.float32) + 0.5) * inv91).astype(jnp.int32)
            pb[pl.ds(j * LANES, LANES)] = q
            return 0
        lax.fori_loop(0, NQ // LANES, _qidx, 0)

        pltpu.sync_copy(kb.at[pl.ds(0, NQ)], score_hbm.at[pl.ds(b * NQ, NQ)])
        pltpu.sync_copy(pb.at[pl.ds(0, NQ)], qidx_hbm.at[pl.ds(b * NQ, NQ)])


SEG = 5000


@functools.partial(
    pl.kernel,
    out_type=jax.ShapeDtypeStruct((BS * 4 * NQ,), jnp.float32),
    mesh=_mesh,
    compiler_params=_sc_params,
    scratch_types=[
        pltpu.VMEM((4 * NQ,), jnp.float32),
        pltpu.VMEM((NQ,), jnp.int32),
        pltpu.VMEM((4 * SEG,), jnp.float32),
        pltpu.VMEM((64,), jnp.float32),
    ],
)
def _boxes_kernel(boxes_hbm, qidx_hbm, scale_hbm, out_hbm,
                  tbl, qv, obuf, ssc):
    c = lax.axis_index("c")
    s = lax.axis_index("s")
    active = s < 8
    b = c * 8 + jnp.where(active, s, 0)
    iota = lax.iota(jnp.int32, LANES)

    @pl.when(active)
    def _():
        pltpu.sync_copy(boxes_hbm.at[pl.ds(b * 4 * NQ, 4 * NQ)], tbl)
        pltpu.sync_copy(qidx_hbm.at[pl.ds(b * NQ, NQ)], qv)
        pltpu.sync_copy(scale_hbm, ssc)
        sv = plsc.load_gather(ssc, [4 * b + (iota & 3)])
        half = jnp.where((iota & 2) == 0, jnp.float32(-0.5), jnp.float32(0.5))
        rep4 = iota // 4
        coord = iota & 3
        shuf_a = iota - (iota & 2)

        for seg in range(NQ // SEG):
            def _one(j, _):
                qq = jnp.take(qv[pl.ds(seg * SEG + 4 * j, LANES)], rep4)
                g = plsc.load_gather(tbl, [4 * qq + coord])
                cxy = jnp.take(g, shuf_a)
                wh = jnp.take(g, shuf_a + 2)
                obuf[pl.ds(j * LANES, LANES)] = (cxy + half * wh) * sv
                return 0
            lax.fori_loop(0, 4 * SEG // LANES, _one, 0)
            pltpu.sync_copy(
                obuf, out_hbm.at[pl.ds(b * 4 * NQ + seg * 4 * SEG, 4 * SEG)])


def kernel(pred_logits, pred_boxes, target_sizes):
    logits3 = pred_logits.reshape(BS, NQ, NC)
    prob = _sigmoid_pad(logits3).reshape(BS * NF)
    ckey, cidx, cnts = _select_kernel(prob)
    score_bits, qidx = _sort_kernel(ckey, cidx, cnts)
    scores = lax.bitcast_convert_type(score_bits, jnp.float32).reshape(BS, NQ)
    labels = jnp.ones((BS, NQ), jnp.int32)
    img_h = target_sizes[:, 0]
    img_w = target_sizes[:, 1]
    scale_fct = jnp.stack([img_w, img_h, img_w, img_h], axis=1)
    boxes = _boxes_kernel(pred_boxes.reshape(BS * 4 * NQ), qidx,
                          scale_fct.reshape(-1))
    return scores, labels, boxes.reshape(BS, NQ, 4)
